# baseline (speedup 1.0000x reference)
import functools

import jax
import jax.numpy as jnp
from jax import lax
from jax.experimental import pallas as pl
from jax.experimental.pallas import tpu as pltpu
from jax.experimental.pallas import tpu_sc as plsc

B = 8
D = 64
N = 2048
K = 20
NK = N * K
NTILES = 32
QP = NK // 4
CNT = float(B * N * K * D)


RB = 512


def _topk_body(xr_ref, xc_ref, idx_ref):
    xr = xr_ref[0]
    xc = xc_ref[0]
    m = lax.dot_general(xr, xc, (((0,), (0,)), ((), ())),
                        preferred_element_type=jnp.float32)
    inner = -2.0 * m
    xx_c = jnp.sum(xc * xc, axis=0, keepdims=True)
    ones = jnp.ones((D, 1), dtype=jnp.float32)
    xx_r = lax.dot_general(xr * xr, ones, (((0,), (0,)), ((), ())),
                           preferred_element_type=jnp.float32)
    dist = (-xx_c - inner) - xx_r
    lane = lax.broadcasted_iota(jnp.int32, (RB, N), 1)
    iota_col = lax.broadcasted_iota(jnp.int32, (N, 1), 0).astype(jnp.float32)
    w = jnp.concatenate(
        [jnp.ones((N, 1), jnp.float32), iota_col, iota_col * iota_col], axis=1)
    cols = []
    for _ in range(K):
        mx = jnp.max(dist, axis=1, keepdims=True)
        eqf = jnp.where(dist == mx, 1.0, 0.0)
        sums = lax.dot_general(eqf, w, (((1,), (0,)), ((), ())),
                               preferred_element_type=jnp.float32,
                               precision=lax.Precision.HIGHEST)
        cnt = sums[:, 0:1]
        s1 = sums[:, 1:2]
        s2 = sums[:, 2:3]
        disc = jnp.maximum(2.0 * s2 - s1 * s1, 0.0)
        amin2 = (s1 - jnp.sqrt(disc)) * 0.5
        aminf = jnp.where(cnt == 2.0, amin2, s1)
        amin = jnp.clip((aminf + 0.5).astype(jnp.int32), 0, N - 1)
        cols.append(amin)
        dist = jnp.where(lane == amin, -jnp.inf, dist)
    idx_ref[0, :, 0:K] = jnp.concatenate(cols, axis=1)


def _topk(x, interpret=False):
    return pl.pallas_call(
        _topk_body,
        grid=(B, N // RB),
        in_specs=[
            pl.BlockSpec((1, D, RB), lambda b, nb: (b, 0, nb)),
            pl.BlockSpec((1, D, N), lambda b, nb: (b, 0, 0)),
        ],
        out_specs=pl.BlockSpec((1, RB, 128), lambda b, nb: (b, nb, 0)),
        out_shape=jax.ShapeDtypeStruct((B, N, 128), jnp.int32),
        interpret=interpret,
    )(x, x)


NQ = N // 4


def _gather_body(x_hbm, idx_hbm, feat_hbm, xrep_hbm, stats_hbm,
                 idx2_v, ilin_v, nidx_v, xc0_v, xc1_v,
                 feat0_v, feat1_v, xrep0_v, xrep1_v, st_v,
                 seml0, seml1, semf0, semf1, semx0, semx1):
    nc = 2
    wid = lax.axis_index("s") * nc + lax.axis_index("c")
    b = lax.shift_right_logical(wid, 2)
    q = lax.bitwise_and(wid, 3)
    base = q * QP
    nq0 = q * NQ

    pltpu.async_copy(x_hbm.at[b, 0], xc0_v, seml0)
    pltpu.async_copy(x_hbm.at[b, 1], xc1_v, seml1)

    lane16 = lax.broadcasted_iota(jnp.int32, (16,), 0)

    for h in range(2):
        row0 = nq0 + h * (NQ // 2)
        pltpu.sync_copy(idx_hbm.at[b, pl.ds(row0, NQ // 2), :], idx2_v)
        hbase = h * (QP // 2)

        def pre_body(p, _):
            pos = base + hbase + p * 16 + lane16
            n16 = lax.shift_right_logical(pos * 52429, 20)
            j16 = pos - n16 * K
            nidx_v[pl.ds(hbase + p * 16, 16)] = n16
            ilin_v[pl.ds(hbase + p * 16, 16)] = plsc.load_gather(
                idx2_v, [n16 - row0, j16])
            return 0

        lax.fori_loop(0, QP // 32, pre_body, 0)

    def chan_compute(xc_v, feat_v, xrep_v, carry):
        def pos_body(p, carry2):
            s2, ss2 = carry2
            o = p * 32
            ia = ilin_v[pl.ds(o, 16)]
            ib = ilin_v[pl.ds(o + 16, 16)]
            na = nidx_v[pl.ds(o, 16)]
            nb = nidx_v[pl.ds(o + 16, 16)]
            g1a = plsc.load_gather(xc_v, [ia])
            g2a = plsc.load_gather(xc_v, [na])
            g1b = plsc.load_gather(xc_v, [ib])
            g2b = plsc.load_gather(xc_v, [nb])
            da = g1a - g2a
            db = g1b - g2b
            feat_v[pl.ds(o, 16)] = da
            feat_v[pl.ds(o + 16, 16)] = db
            xrep_v[pl.ds(o, 16)] = g2a
            xrep_v[pl.ds(o + 16, 16)] = g2b
            return (s2 + da + db, ss2 + da * da + db * db)

        return lax.fori_loop(0, QP // 32, pos_body, carry)

    def half(g, c, carry, xc_v, feat_v, xrep_v, seml, semf, semx, c_next,
             xcn_v, semln):
        pltpu.make_async_copy(x_hbm.at[b, 0], xc_v, seml).wait()

        @pl.when(g > 0)
        def _():
            pltpu.make_async_copy(feat_v, feat_hbm.at[b, 0, pl.ds(0, QP)],
                                  semf).wait()
            pltpu.make_async_copy(xrep_v, xrep_hbm.at[b, 0, pl.ds(0, QP)],
                                  semx).wait()

        carry = chan_compute(xc_v, feat_v, xrep_v, carry)
        pltpu.async_copy(feat_v, feat_hbm.at[b, c, pl.ds(base, QP)], semf)
        pltpu.async_copy(xrep_v, xrep_hbm.at[b, c, pl.ds(base, QP)], semx)
        pltpu.async_copy(x_hbm.at[b, c_next], xcn_v, semln)
        return carry

    def chan_body(g, carry):
        c0 = 2 * g
        c1 = 2 * g + 1
        carry = half(g, c0, carry, xc0_v, feat0_v, xrep0_v,
                     seml0, semf0, semx0, jnp.minimum(c0 + 2, D - 1),
                     xc0_v, seml0)
        carry = half(g, c1, carry, xc1_v, feat1_v, xrep1_v,
                     seml1, semf1, semx1, jnp.minimum(c1 + 2, D - 1),
                     xc1_v, seml1)
        return carry

    zero = jnp.zeros((16,), jnp.float32)
    s, ss = lax.fori_loop(0, D // 2, chan_body, (zero, zero))
    pltpu.make_async_copy(feat0_v, feat_hbm.at[b, 0, pl.ds(0, QP)], semf0).wait()
    pltpu.make_async_copy(xrep0_v, xrep_hbm.at[b, 0, pl.ds(0, QP)], semx0).wait()
    pltpu.make_async_copy(feat1_v, feat_hbm.at[b, 0, pl.ds(0, QP)], semf1).wait()
    pltpu.make_async_copy(xrep1_v, xrep_hbm.at[b, 0, pl.ds(0, QP)], semx1).wait()
    pltpu.make_async_copy(x_hbm.at[b, 0], xc0_v, seml0).wait()
    pltpu.make_async_copy(x_hbm.at[b, 0], xc1_v, seml1).wait()
    st_v[pl.ds(0, 16)] = s
    st_v[pl.ds(16, 16)] = ss
    pltpu.sync_copy(st_v, stats_hbm.at[wid])


def _gather(x, idx):
    mesh = plsc.VectorSubcoreMesh(core_axis_name="c", subcore_axis_name="s")
    fn = pl.kernel(
        _gather_body,
        out_type=(
            jax.ShapeDtypeStruct((B, D, NK), jnp.float32),
            jax.ShapeDtypeStruct((B, D, NK), jnp.float32),
            jax.ShapeDtypeStruct((NTILES, 32), jnp.float32),
        ),
        mesh=mesh,
        compiler_params=pltpu.CompilerParams(needs_layout_passes=False),
        scratch_types=[
            pltpu.VMEM((NQ // 2, 128), jnp.int32),
            pltpu.VMEM((QP,), jnp.int32),
            pltpu.VMEM((QP,), jnp.int32),
            pltpu.VMEM((N,), jnp.float32),
            pltpu.VMEM((N,), jnp.float32),
            pltpu.VMEM((QP,), jnp.float32),
            pltpu.VMEM((QP,), jnp.float32),
            pltpu.VMEM((QP,), jnp.float32),
            pltpu.VMEM((QP,), jnp.float32),
            pltpu.VMEM((32,), jnp.float32),
            pltpu.SemaphoreType.DMA,
            pltpu.SemaphoreType.DMA,
            pltpu.SemaphoreType.DMA,
            pltpu.SemaphoreType.DMA,
            pltpu.SemaphoreType.DMA,
            pltpu.SemaphoreType.DMA,
        ],
    )
    return fn(x, idx)


LNB = 2560


def _final_body(d_ref, xr_ref, st_ref, a_ref, b_ref, out_ref):
    s = jnp.sum(st_ref[:, 0:16])
    ss = jnp.sum(st_ref[:, 16:32])
    var = (ss - s * s / CNT) / (CNT - 1.0)
    inv = 1.0 / (jnp.sqrt(var) + 1e-5)
    alpha = a_ref[...]
    beta = b_ref[...]
    out_ref[0, 0:D, :] = alpha * (d_ref[0] * inv) + beta
    out_ref[0, D:2 * D, :] = xr_ref[0]


def _finalize(feat, xrep, stats, alpha, beta, interpret=False):
    return pl.pallas_call(
        _final_body,
        grid=(B, NK // LNB),
        in_specs=[
            pl.BlockSpec((1, D, LNB), lambda b, l: (b, 0, l)),
            pl.BlockSpec((1, D, LNB), lambda b, l: (b, 0, l)),
            pl.BlockSpec((NTILES, 32), lambda b, l: (0, 0)),
            pl.BlockSpec((D, 1), lambda b, l: (0, 0)),
            pl.BlockSpec((D, 1), lambda b, l: (0, 0)),
        ],
        out_specs=pl.BlockSpec((1, 2 * D, LNB), lambda b, l: (b, 0, l)),
        out_shape=jax.ShapeDtypeStruct((B, 2 * D, NK), jnp.float32),
        interpret=interpret,
    )(feat, xrep, stats, alpha, beta)


def kernel(x, affine_alpha, affine_beta):
    idx = _topk(x)
    feat, xrep, stats = _gather(x, idx)
    a_col = affine_alpha.reshape(D, 1)
    b_col = affine_beta.reshape(D, 1)
    out = _finalize(feat, xrep, stats, a_col, b_col)
    return out.reshape(B, 2 * D, N, K)


# speedup vs baseline: 2.3111x; 2.3111x over previous
import functools

import jax
import jax.numpy as jnp
from jax import lax
from jax.experimental import pallas as pl
from jax.experimental.pallas import tpu as pltpu
from jax.experimental.pallas import tpu_sc as plsc

B = 8
D = 64
N = 2048
K = 20
NK = N * K
NTILES = 32
QP = NK // 4
CNT = float(B * N * K * D)


RB = 512


def _topk_body(xr_ref, xc_ref, idx_ref):
    xr = xr_ref[0]
    xc = xc_ref[0]
    m = lax.dot_general(xr, xc, (((0,), (0,)), ((), ())),
                        preferred_element_type=jnp.float32)
    inner = -2.0 * m
    xx_c = jnp.sum(xc * xc, axis=0, keepdims=True)
    ones = jnp.ones((D, 1), dtype=jnp.float32)
    xx_r = lax.dot_general(xr * xr, ones, (((0,), (0,)), ((), ())),
                           preferred_element_type=jnp.float32)
    dist = (-xx_c - inner) - xx_r
    lane = lax.broadcasted_iota(jnp.int32, (RB, N), 1)
    ii = lax.broadcasted_iota(jnp.int32, (N, 1), 0)
    h = lax.shift_right_logical(ii, 7)
    m = lax.bitwise_and(lax.shift_right_logical(ii, 3), 15)
    l = lax.bitwise_and(ii, 7)
    w = jnp.concatenate(
        [jnp.ones((N, 1), jnp.float32)] +
        [c.astype(jnp.float32)
         for c in (h, m, l, h * h, m * m, l * l, h * m, h * l, m * l)],
        axis=1)
    cols = []
    for _ in range(K):
        mx = jnp.max(dist, axis=1, keepdims=True)
        eqf = jnp.where(dist == mx, 1.0, 0.0)
        d = lax.dot_general(eqf, w, (((1,), (0,)), ((), ())),
                            preferred_element_type=jnp.float32)
        cnt = d[:, 0:1]
        s1 = 128.0 * d[:, 1:2] + 8.0 * d[:, 2:3] + d[:, 3:4]
        s2 = (16384.0 * d[:, 4:5] + 64.0 * d[:, 5:6] + d[:, 6:7] +
              2048.0 * d[:, 7:8] + 256.0 * d[:, 8:9] + 16.0 * d[:, 9:10])
        disc = jnp.maximum(2.0 * s2 - s1 * s1, 0.0)
        amin2 = (s1 - jnp.sqrt(disc)) * 0.5
        aminf = jnp.where(cnt == 2.0, amin2, s1)
        amin = jnp.clip((aminf + 0.5).astype(jnp.int32), 0, N - 1)
        cols.append(amin)
        dist = jnp.where(lane == amin, -jnp.inf, dist)
    idx_ref[0, :, 0:K] = jnp.concatenate(cols, axis=1)


def _topk(x, interpret=False):
    return pl.pallas_call(
        _topk_body,
        grid=(B, N // RB),
        in_specs=[
            pl.BlockSpec((1, D, RB), lambda b, nb: (b, 0, nb)),
            pl.BlockSpec((1, D, N), lambda b, nb: (b, 0, 0)),
        ],
        out_specs=pl.BlockSpec((1, RB, 128), lambda b, nb: (b, nb, 0)),
        out_shape=jax.ShapeDtypeStruct((B, N, 128), jnp.int32),
        interpret=interpret,
    )(x, x)


NQ = N // 4


def _gather_body(x_hbm, idx_hbm, feat_hbm, xrep_hbm, stats_hbm,
                 idx2_v, ilin_v, nidx_v, xc0_v, xc1_v,
                 feat0_v, feat1_v, xrep0_v, xrep1_v, st_v,
                 seml0, seml1, semf0, semf1, semx0, semx1):
    nc = 2
    wid = lax.axis_index("s") * nc + lax.axis_index("c")
    b = lax.shift_right_logical(wid, 2)
    q = lax.bitwise_and(wid, 3)
    base = q * QP
    nq0 = q * NQ

    pltpu.async_copy(x_hbm.at[b, 0], xc0_v, seml0)
    pltpu.async_copy(x_hbm.at[b, 1], xc1_v, seml1)

    lane16 = lax.broadcasted_iota(jnp.int32, (16,), 0)

    for h in range(2):
        row0 = nq0 + h * (NQ // 2)
        pltpu.sync_copy(idx_hbm.at[b, pl.ds(row0, NQ // 2), :], idx2_v)
        hbase = h * (QP // 2)

        def pre_body(p, _):
            pos = base + hbase + p * 16 + lane16
            n16 = lax.shift_right_logical(pos * 52429, 20)
            j16 = pos - n16 * K
            nidx_v[pl.ds(hbase + p * 16, 16)] = n16
            ilin_v[pl.ds(hbase + p * 16, 16)] = plsc.load_gather(
                idx2_v, [n16 - row0, j16])
            return 0

        lax.fori_loop(0, QP // 32, pre_body, 0)

    def chan_compute(xc_v, feat_v, xrep_v, carry):
        def pos_body(p, carry2):
            s2, ss2 = carry2
            o = p * 32
            ia = ilin_v[pl.ds(o, 16)]
            ib = ilin_v[pl.ds(o + 16, 16)]
            na = nidx_v[pl.ds(o, 16)]
            nb = nidx_v[pl.ds(o + 16, 16)]
            g1a = plsc.load_gather(xc_v, [ia])
            g2a = plsc.load_gather(xc_v, [na])
            g1b = plsc.load_gather(xc_v, [ib])
            g2b = plsc.load_gather(xc_v, [nb])
            da = g1a - g2a
            db = g1b - g2b
            feat_v[pl.ds(o, 16)] = da
            feat_v[pl.ds(o + 16, 16)] = db
            xrep_v[pl.ds(o, 16)] = g2a
            xrep_v[pl.ds(o + 16, 16)] = g2b
            return (s2 + da + db, ss2 + da * da + db * db)

        return lax.fori_loop(0, QP // 32, pos_body, carry)

    def half(g, c, carry, xc_v, feat_v, xrep_v, seml, semf, semx, c_next,
             xcn_v, semln):
        pltpu.make_async_copy(x_hbm.at[b, 0], xc_v, seml).wait()

        @pl.when(g > 0)
        def _():
            pltpu.make_async_copy(feat_v, feat_hbm.at[b, 0, pl.ds(0, QP)],
                                  semf).wait()
            pltpu.make_async_copy(xrep_v, xrep_hbm.at[b, 0, pl.ds(0, QP)],
                                  semx).wait()

        carry = chan_compute(xc_v, feat_v, xrep_v, carry)
        pltpu.async_copy(feat_v, feat_hbm.at[b, c, pl.ds(base, QP)], semf)
        pltpu.async_copy(xrep_v, xrep_hbm.at[b, c, pl.ds(base, QP)], semx)
        pltpu.async_copy(x_hbm.at[b, c_next], xcn_v, semln)
        return carry

    def chan_body(g, carry):
        c0 = 2 * g
        c1 = 2 * g + 1
        carry = half(g, c0, carry, xc0_v, feat0_v, xrep0_v,
                     seml0, semf0, semx0, jnp.minimum(c0 + 2, D - 1),
                     xc0_v, seml0)
        carry = half(g, c1, carry, xc1_v, feat1_v, xrep1_v,
                     seml1, semf1, semx1, jnp.minimum(c1 + 2, D - 1),
                     xc1_v, seml1)
        return carry

    zero = jnp.zeros((16,), jnp.float32)
    s, ss = lax.fori_loop(0, D // 2, chan_body, (zero, zero))
    pltpu.make_async_copy(feat0_v, feat_hbm.at[b, 0, pl.ds(0, QP)], semf0).wait()
    pltpu.make_async_copy(xrep0_v, xrep_hbm.at[b, 0, pl.ds(0, QP)], semx0).wait()
    pltpu.make_async_copy(feat1_v, feat_hbm.at[b, 0, pl.ds(0, QP)], semf1).wait()
    pltpu.make_async_copy(xrep1_v, xrep_hbm.at[b, 0, pl.ds(0, QP)], semx1).wait()
    pltpu.make_async_copy(x_hbm.at[b, 0], xc0_v, seml0).wait()
    pltpu.make_async_copy(x_hbm.at[b, 0], xc1_v, seml1).wait()
    st_v[pl.ds(0, 16)] = s
    st_v[pl.ds(16, 16)] = ss
    pltpu.sync_copy(st_v, stats_hbm.at[wid])


def _gather(x, idx):
    mesh = plsc.VectorSubcoreMesh(core_axis_name="c", subcore_axis_name="s")
    fn = pl.kernel(
        _gather_body,
        out_type=(
            jax.ShapeDtypeStruct((B, D, NK), jnp.float32),
            jax.ShapeDtypeStruct((B, D, NK), jnp.float32),
            jax.ShapeDtypeStruct((NTILES, 32), jnp.float32),
        ),
        mesh=mesh,
        compiler_params=pltpu.CompilerParams(needs_layout_passes=False),
        scratch_types=[
            pltpu.VMEM((NQ // 2, 128), jnp.int32),
            pltpu.VMEM((QP,), jnp.int32),
            pltpu.VMEM((QP,), jnp.int32),
            pltpu.VMEM((N,), jnp.float32),
            pltpu.VMEM((N,), jnp.float32),
            pltpu.VMEM((QP,), jnp.float32),
            pltpu.VMEM((QP,), jnp.float32),
            pltpu.VMEM((QP,), jnp.float32),
            pltpu.VMEM((QP,), jnp.float32),
            pltpu.VMEM((32,), jnp.float32),
            pltpu.SemaphoreType.DMA,
            pltpu.SemaphoreType.DMA,
            pltpu.SemaphoreType.DMA,
            pltpu.SemaphoreType.DMA,
            pltpu.SemaphoreType.DMA,
            pltpu.SemaphoreType.DMA,
        ],
    )
    return fn(x, idx)


LNB = 2560


def _final_body(d_ref, xr_ref, st_ref, a_ref, b_ref, out_ref):
    s = jnp.sum(st_ref[:, 0:16])
    ss = jnp.sum(st_ref[:, 16:32])
    var = (ss - s * s / CNT) / (CNT - 1.0)
    inv = 1.0 / (jnp.sqrt(var) + 1e-5)
    alpha = a_ref[...]
    beta = b_ref[...]
    out_ref[0, 0:D, :] = alpha * (d_ref[0] * inv) + beta
    out_ref[0, D:2 * D, :] = xr_ref[0]


def _finalize(feat, xrep, stats, alpha, beta, interpret=False):
    return pl.pallas_call(
        _final_body,
        grid=(B, NK // LNB),
        in_specs=[
            pl.BlockSpec((1, D, LNB), lambda b, l: (b, 0, l)),
            pl.BlockSpec((1, D, LNB), lambda b, l: (b, 0, l)),
            pl.BlockSpec((NTILES, 32), lambda b, l: (0, 0)),
            pl.BlockSpec((D, 1), lambda b, l: (0, 0)),
            pl.BlockSpec((D, 1), lambda b, l: (0, 0)),
        ],
        out_specs=pl.BlockSpec((1, 2 * D, LNB), lambda b, l: (b, 0, l)),
        out_shape=jax.ShapeDtypeStruct((B, 2 * D, NK), jnp.float32),
        interpret=interpret,
    )(feat, xrep, stats, alpha, beta)


def kernel(x, affine_alpha, affine_beta):
    idx = _topk(x)
    feat, xrep, stats = _gather(x, idx)
    a_col = affine_alpha.reshape(D, 1)
    b_col = affine_beta.reshape(D, 1)
    out = _finalize(feat, xrep, stats, a_col, b_col)
    return out.reshape(B, 2 * D, N, K)


# speedup vs baseline: 5.5584x; 2.4051x over previous
import functools

import jax
import jax.numpy as jnp
from jax import lax
from jax.experimental import pallas as pl
from jax.experimental.pallas import tpu as pltpu
from jax.experimental.pallas import tpu_sc as plsc

B = 8
D = 64
N = 2048
K = 20
NK = N * K
NTILES = 32
QP = NK // 4
CNT = float(B * N * K * D)


RB = 512


def _topk_body(xr_ref, xc_ref, idx_ref):
    xr = xr_ref[0]
    xc = xc_ref[0]
    m = lax.dot_general(xr, xc, (((0,), (0,)), ((), ())),
                        preferred_element_type=jnp.float32)
    inner = -2.0 * m
    xx_c = jnp.sum(xc * xc, axis=0, keepdims=True)
    ones = jnp.ones((D, 1), dtype=jnp.float32)
    xx_r = lax.dot_general(xr * xr, ones, (((0,), (0,)), ((), ())),
                           preferred_element_type=jnp.float32)
    dist = (-xx_c - inner) - xx_r
    lane = lax.broadcasted_iota(jnp.int32, (RB, N), 1)
    cols = []
    for _ in range(K):
        mx = jnp.max(dist, axis=1, keepdims=True)
        cand = jnp.where(dist == mx, lane, N)
        amin = jnp.min(cand, axis=1, keepdims=True)
        cols.append(amin)
        dist = jnp.where(lane == amin, -jnp.inf, dist)
    idx_ref[0, :, 0:K] = jnp.concatenate(cols, axis=1)


def _topk(x, interpret=False):
    return pl.pallas_call(
        _topk_body,
        grid=(B, N // RB),
        in_specs=[
            pl.BlockSpec((1, D, RB), lambda b, nb: (b, 0, nb)),
            pl.BlockSpec((1, D, N), lambda b, nb: (b, 0, 0)),
        ],
        out_specs=pl.BlockSpec((1, RB, 128), lambda b, nb: (b, nb, 0)),
        out_shape=jax.ShapeDtypeStruct((B, N, 128), jnp.int32),
        interpret=interpret,
    )(x, x)


NQ = N // 4


def _gather_body(x_hbm, idx_hbm, feat_hbm, stats_hbm,
                 idx2_v, ilin_v, xc0_v, xc1_v,
                 feat0_v, feat1_v, st_v,
                 seml0, seml1, semf0, semf1):
    nc = 2
    wid = lax.axis_index("s") * nc + lax.axis_index("c")
    b = lax.shift_right_logical(wid, 2)
    q = lax.bitwise_and(wid, 3)
    nq0 = q * NQ

    pltpu.async_copy(x_hbm.at[b, 0], xc0_v, seml0)
    pltpu.async_copy(x_hbm.at[b, 1], xc1_v, seml1)

    lane16 = lax.broadcasted_iota(jnp.int32, (16,), 0)

    pltpu.sync_copy(idx_hbm.at[b, pl.ds(nq0, NQ), :], idx2_v)

    def pre_body(t, _):
        j = lax.shift_right_logical(t, 5)
        ch = lax.bitwise_and(t, 31)
        nl16 = ch * 16 + lane16
        j16 = jnp.full((16,), 0, jnp.int32) + j
        ilin_v[pl.ds(t * 16, 16)] = plsc.load_gather(idx2_v, [nl16, j16])
        return 0

    lax.fori_loop(0, QP // 16, pre_body, 0)

    def chan_compute(xc_v, feat_v, carry):
        def pos_body(p, carry2):
            s2, ss2 = carry2
            o = p * 32
            row = lax.shift_right_logical(o, 9)
            col = lax.bitwise_and(o, 511)
            ia = ilin_v[pl.ds(o, 16)]
            ib = ilin_v[pl.ds(o + 16, 16)]
            g1a = plsc.load_gather(xc_v, [ia])
            g1b = plsc.load_gather(xc_v, [ib])
            xna = xc_v[pl.ds(nq0 + col, 16)]
            xnb = xc_v[pl.ds(nq0 + col + 16, 16)]
            da = g1a - xna
            db = g1b - xnb
            feat_v[row, pl.ds(col, 16)] = da
            feat_v[row, pl.ds(col + 16, 16)] = db
            return (s2 + da + db, ss2 + da * da + db * db)

        return lax.fori_loop(0, QP // 32, pos_body, carry)

    def half(g, c, carry, xc_v, feat_v, seml, semf, c_next):
        pltpu.make_async_copy(x_hbm.at[b, 0], xc_v, seml).wait()

        @pl.when(g > 0)
        def _():
            pltpu.make_async_copy(
                feat_v, feat_hbm.at[b, :, 0, pl.ds(0, NQ)], semf).wait()

        carry = chan_compute(xc_v, feat_v, carry)
        pltpu.async_copy(feat_v, feat_hbm.at[b, :, c, pl.ds(nq0, NQ)], semf)
        pltpu.async_copy(x_hbm.at[b, c_next], xc_v, seml)
        return carry

    def chan_body(g, carry):
        c0 = 2 * g
        c1 = 2 * g + 1
        carry = half(g, c0, carry, xc0_v, feat0_v, seml0, semf0,
                     jnp.minimum(c0 + 2, D - 1))
        carry = half(g, c1, carry, xc1_v, feat1_v, seml1, semf1,
                     jnp.minimum(c1 + 2, D - 1))
        return carry

    zero = jnp.zeros((16,), jnp.float32)
    s, ss = lax.fori_loop(0, D // 2, chan_body, (zero, zero))
    pltpu.make_async_copy(feat0_v, feat_hbm.at[b, :, 0, pl.ds(0, NQ)], semf0).wait()
    pltpu.make_async_copy(feat1_v, feat_hbm.at[b, :, 0, pl.ds(0, NQ)], semf1).wait()
    pltpu.make_async_copy(x_hbm.at[b, 0], xc0_v, seml0).wait()
    pltpu.make_async_copy(x_hbm.at[b, 0], xc1_v, seml1).wait()
    st_v[pl.ds(0, 16)] = s
    st_v[pl.ds(16, 16)] = ss
    pltpu.sync_copy(st_v, stats_hbm.at[wid])


def _gather(x, idx):
    mesh = plsc.VectorSubcoreMesh(core_axis_name="c", subcore_axis_name="s")
    fn = pl.kernel(
        _gather_body,
        out_type=(
            jax.ShapeDtypeStruct((B, K, D, N), jnp.float32),
            jax.ShapeDtypeStruct((NTILES, 32), jnp.float32),
        ),
        mesh=mesh,
        compiler_params=pltpu.CompilerParams(needs_layout_passes=False),
        scratch_types=[
            pltpu.VMEM((NQ, 128), jnp.int32),
            pltpu.VMEM((QP,), jnp.int32),
            pltpu.VMEM((N,), jnp.float32),
            pltpu.VMEM((N,), jnp.float32),
            pltpu.VMEM((K, NQ), jnp.float32),
            pltpu.VMEM((K, NQ), jnp.float32),
            pltpu.VMEM((32,), jnp.float32),
            pltpu.SemaphoreType.DMA,
            pltpu.SemaphoreType.DMA,
            pltpu.SemaphoreType.DMA,
            pltpu.SemaphoreType.DMA,
        ],
    )
    return fn(x, idx)


NB3 = 512


def _final_body(d_ref, x_ref, st_ref, a_ref, b_ref, out_ref):
    s = jnp.sum(st_ref[:, 0:16])
    ss = jnp.sum(st_ref[:, 16:32])
    var = (ss - s * s / CNT) / (CNT - 1.0)
    inv = 1.0 / (jnp.sqrt(var) + 1e-5)
    alpha = a_ref[...].reshape(1, D, 1)
    beta = b_ref[...].reshape(1, D, 1)
    out_ref[0, :, 0:D, :] = alpha * (d_ref[0] * inv) + beta
    out_ref[0, :, D:2 * D, :] = jnp.broadcast_to(
        x_ref[0][None, :, :], (K, D, NB3))


def _finalize(feat, x, stats, alpha, beta, interpret=False):
    return pl.pallas_call(
        _final_body,
        grid=(B, N // NB3),
        in_specs=[
            pl.BlockSpec((1, K, D, NB3), lambda b, l: (b, 0, 0, l)),
            pl.BlockSpec((1, D, NB3), lambda b, l: (b, 0, l)),
            pl.BlockSpec((NTILES, 32), lambda b, l: (0, 0)),
            pl.BlockSpec((D, 1), lambda b, l: (0, 0)),
            pl.BlockSpec((D, 1), lambda b, l: (0, 0)),
        ],
        out_specs=pl.BlockSpec((1, K, 2 * D, NB3), lambda b, l: (b, 0, 0, l)),
        out_shape=jax.ShapeDtypeStruct((B, K, 2 * D, N), jnp.float32),
        interpret=interpret,
    )(feat, x, stats, alpha, beta)


def kernel(x, affine_alpha, affine_beta):
    idx = _topk(x)
    feat, stats = _gather(x, idx)
    a_col = affine_alpha.reshape(D, 1)
    b_col = affine_beta.reshape(D, 1)
    out = _finalize(feat, x, stats, a_col, b_col)
    return jnp.transpose(out, (0, 2, 3, 1))
